# Initial kernel scaffold; baseline (speedup 1.0000x reference)
import functools
import jax
import jax.numpy as jnp
from jax import lax
import jax.scipy.linalg as jsl
from jax.experimental import pallas as pl
from jax.experimental.pallas import tpu as pltpu
from jax.experimental.pallas import tpu_sc as plsc

N = 50000
E = 800000
VOCAB = 1000
N_ACC = 50048
E_PAD = 819200
CH = 1024
NCHUNK = 50
RB = N_ACC // 16
BN = 1000
F32 = jnp.float32

_sc_mesh = plsc.VectorSubcoreMesh(
    core_axis_name="c", subcore_axis_name="s", num_cores=2, num_subcores=16)


@functools.partial(
    pl.kernel,
    out_type=jax.ShapeDtypeStruct((204800, 16), F32),
    mesh=_sc_mesh,
    scratch_types=[
        pltpu.VMEM((128,), jnp.int32),
        pltpu.VMEM((128, 16), F32),
        pltpu.SemaphoreType.DMA,
    ],
)
def _embed(idx2d, tab, out, idxv, rows, sem):
    c = lax.axis_index("c")
    s = lax.axis_index("s")
    wid = s * 2 + c

    def it(r, carry):
        row = wid * 50 + r
        pltpu.sync_copy(idx2d.at[row], idxv)
        pltpu.async_copy(tab.at[idxv], rows, sem).wait()
        pltpu.sync_copy(rows, out.at[pl.ds(row * 128, 128)])
        return carry

    lax.fori_loop(0, 50, it, 0)


@functools.partial(
    pl.kernel,
    out_type=jax.ShapeDtypeStruct((2, N_ACC, 34), F32),
    mesh=_sc_mesh,
    scratch_types=[
        pltpu.VMEM((8, 128), jnp.int32),
        pltpu.VMEM((8, 128), jnp.int32),
        pltpu.VMEM((CH, 16), F32),
        pltpu.VMEM((CH, 16), F32),
        pltpu.VMEM((CH, 32), F32),
        pltpu.VMEM((CH, 34), F32),
        pltpu.VMEM_SHARED((N_ACC, 34), F32),
        pltpu.SemaphoreType.DMA,
        pltpu.SemaphoreType.DMA,
        pltpu.SemaphoreType.DMA,
    ],
)
def _edge_pass(srcr, dstr, esed, wh2, zsrc, ud,
               idx_s, idx_d, esrows, edrows, whrows, msgal, acc,
               sem_a, sem_b, sem_c):
    c = lax.axis_index("c")
    s = lax.axis_index("s")

    pltpu.sync_copy(zsrc.at[pl.ds(s * RB, RB)], acc.at[pl.ds(s * RB, RB)])
    plsc.subcore_barrier()

    iota = lax.iota(jnp.int32, 16)
    hpar = lax.bitwise_and(iota, 1)
    erow_off = lax.shift_right_logical(iota, 1)
    es_col = 2 * c + hpar
    ed_col = es_col + 4
    al_col = 32 + hpar

    def chunk(ch, carry):
        pltpu.sync_copy(srcr.at[s].at[pl.ds(ch * 8, 8)], idx_s)
        pltpu.sync_copy(dstr.at[s].at[pl.ds(ch * 8, 8)], idx_d)
        descs = []
        for j in range(8):
            descs.append(pltpu.async_copy(
                esed.at[idx_s.at[j]], esrows.at[pl.ds(j * 128, 128)], sem_a))
            descs.append(pltpu.async_copy(
                esed.at[idx_d.at[j]], edrows.at[pl.ds(j * 128, 128)], sem_b))
            descs.append(pltpu.async_copy(
                wh2.at[c].at[idx_s.at[j]], whrows.at[pl.ds(j * 128, 128)], sem_c))
        for d in descs:
            d.wait()

        def group(g, gc):
            rows = g * 8 + erow_off
            es = plsc.load_gather(esrows, [rows, es_col])
            ed = plsc.load_gather(edrows, [rows, ed_col])
            e = es + ed
            e = jnp.maximum(e, 0.2 * e)
            al = jnp.exp(e)
            plsc.store_scatter(msgal, [rows, al_col], al)
            base = g * 8
            for j in range(16):
                el = base + (j >> 1)
                hc = (j & 1) * 16
                a = msgal[el, 32 + (j & 1)]
                msgal[el, pl.ds(hc, 16)] = whrows[el, pl.ds(hc, 16)] * a
            return gc

        lax.fori_loop(0, CH // 8, group, 0)

        for j in range(8):
            pltpu.sync_copy(msgal.at[pl.ds(j * 128, 128)],
                            acc.at[idx_d.at[j]], add=True)
        return carry

    lax.fori_loop(0, NCHUNK, chunk, 0)

    plsc.subcore_barrier()
    pltpu.sync_copy(acc.at[pl.ds(s * RB, RB)], ud.at[c].at[pl.ds(s * RB, RB)])


def _pre_body(xn, cat, wnum, bnum, wina, winb, bin_, w1, ap1,
              h_out, whp, esed_out):
    ne = jnp.maximum(xn[...] @ wnum[...] + bnum[...], 0.)
    h = jnp.maximum(ne @ wina[...] + cat[...] @ winb[...] + bin_[...], 0.)
    h_out[...] = h
    wh = h @ w1[...]
    whp[0] = wh[:, :32]
    whp[1] = wh[:, 32:]
    esed_out[...] = wh @ ap1[...]


def _gat_norm(hv, ud_, g, b):
    parts = []
    for cc in range(2):
        for hh in range(2):
            u = ud_[cc, :, hh * 16:(hh + 1) * 16]
            d = jnp.maximum(ud_[cc, :, 32 + hh:33 + hh], 1e-8)
            parts.append(u / d)
    gat = jnp.concatenate(parts, axis=1)
    y = hv + gat
    m = jnp.mean(y, axis=1, keepdims=True)
    v = jnp.mean((y - m) ** 2, axis=1, keepdims=True)
    yl = (y - m) / jnp.sqrt(v + 1e-5) * g + b
    return jnp.maximum(yl, 0.)


def _mid_body(h, ud, g1, b1, w2, ap2, hmid_out, whp2, esed2):
    hm = _gat_norm(h[...], ud[...], g1[...], b1[...])
    hmid_out[...] = hm
    wh = hm @ w2[...]
    whp2[0] = wh[:, :32]
    whp2[1] = wh[:, 32:]
    esed2[...] = wh @ ap2[...]


def _post_body(h, ud, g2, b2, wop, bop, out):
    hf = _gat_norm(h[...], ud[...], g2[...], b2[...])
    out[...] = hf @ wop[...] + bop[...]


def _wspec(shape):
    nd = len(shape)
    return pl.BlockSpec(shape, lambda i: (0,) * nd)


def _stage_pre(x_num, cat, wnum, bnum, wina, winb, bin_, w1, ap1):
    return pl.pallas_call(
        _pre_body,
        grid=(N // BN,),
        in_specs=[
            pl.BlockSpec((BN, 16), lambda i: (i, 0)),
            pl.BlockSpec((BN, 64), lambda i: (i, 0)),
            _wspec((16, 16)), _wspec((1, 16)), _wspec((16, 64)),
            _wspec((64, 64)), _wspec((1, 64)), _wspec((64, 64)),
            _wspec((64, 16)),
        ],
        out_specs=[
            pl.BlockSpec((BN, 64), lambda i: (i, 0)),
            pl.BlockSpec((2, BN, 32), lambda i: (0, i, 0)),
            pl.BlockSpec((BN, 16), lambda i: (i, 0)),
        ],
        out_shape=[
            jax.ShapeDtypeStruct((N, 64), F32),
            jax.ShapeDtypeStruct((2, N, 32), F32),
            jax.ShapeDtypeStruct((N, 16), F32),
        ],
    )(x_num, cat, wnum, bnum, wina, winb, bin_, w1, ap1)


def _stage_mid(h, ud, g1, b1, w2, ap2):
    return pl.pallas_call(
        _mid_body,
        grid=(N // BN,),
        in_specs=[
            pl.BlockSpec((BN, 64), lambda i: (i, 0)),
            pl.BlockSpec((2, BN, 34), lambda i: (0, i, 0)),
            _wspec((1, 64)), _wspec((1, 64)), _wspec((64, 64)),
            _wspec((64, 16)),
        ],
        out_specs=[
            pl.BlockSpec((BN, 64), lambda i: (i, 0)),
            pl.BlockSpec((2, BN, 32), lambda i: (0, i, 0)),
            pl.BlockSpec((BN, 16), lambda i: (i, 0)),
        ],
        out_shape=[
            jax.ShapeDtypeStruct((N, 64), F32),
            jax.ShapeDtypeStruct((2, N, 32), F32),
            jax.ShapeDtypeStruct((N, 16), F32),
        ],
    )(h, ud, g1, b1, w2, ap2)


def _stage_post(h, ud, g2, b2, wop, bop):
    return pl.pallas_call(
        _post_body,
        grid=(N // BN,),
        in_specs=[
            pl.BlockSpec((BN, 64), lambda i: (i, 0)),
            pl.BlockSpec((2, BN, 34), lambda i: (0, i, 0)),
            _wspec((1, 64)), _wspec((1, 64)), _wspec((64, 8)),
            _wspec((1, 8)),
        ],
        out_specs=pl.BlockSpec((BN, 8), lambda i: (i, 0)),
        out_shape=jax.ShapeDtypeStruct((N, 8), F32),
    )(h, ud, g2, b2, wop, bop)


def _apack(a_src, a_dst):
    bs = jsl.block_diag(*[a_src[h][:, None] for h in range(4)])
    bd = jsl.block_diag(*[a_dst[h][:, None] for h in range(4)])
    return jnp.concatenate([bs, bd, jnp.zeros((64, 8), F32)], axis=1)


@jax.jit
def kernel(x_num, x_cat, edge_index, W_num, b_num, E0, E1, E2, E3, W_in, b_in,
           W1, a_src1, a_dst1, W2, a_src2, a_dst2, g1, be1, g2, be2,
           W_out, b_out):
    x_num = x_num.astype(F32)
    xc = jnp.clip(x_cat, 0, VOCAB).astype(jnp.int32)
    idx_flat = (xc + jnp.arange(4, dtype=jnp.int32) * (VOCAB + 1)).reshape(-1)
    idx_pad = jnp.concatenate(
        [idx_flat, jnp.zeros((204800 - 4 * N,), jnp.int32)]).reshape(1600, 128)
    Ecat = jnp.concatenate([E0, E1, E2, E3], axis=0)
    emb = _embed(idx_pad, Ecat)
    cat_embed = emb[:4 * N].reshape(N, 64)

    src = edge_index[0].astype(jnp.int32)
    dst = edge_index[1].astype(jnp.int32)
    pe = E_PAD - E
    srcr = jnp.concatenate([src, jnp.zeros((pe,), jnp.int32)]).reshape(16, 400, 128)
    dstr = jnp.concatenate([dst, jnp.full((pe,), N, jnp.int32)]).reshape(16, 400, 128)
    zsrc = jnp.zeros((N_ACC, 34), F32)
    zpad = jnp.zeros((N_ACC - N, 16), F32)

    ap1 = _apack(a_src1, a_dst1)
    ap2 = _apack(a_src2, a_dst2)

    h, whp1, esed1 = _stage_pre(
        x_num, cat_embed, W_num, b_num.reshape(1, 16), W_in[:16], W_in[16:],
        b_in.reshape(1, 64), W1, ap1)
    ud1 = _edge_pass(srcr, dstr, jnp.concatenate([esed1, zpad]), whp1, zsrc)
    hmid, whp2, esed2 = _stage_mid(
        h, ud1[:, :N], g1.reshape(1, 64), be1.reshape(1, 64), W2, ap2)
    ud2 = _edge_pass(srcr, dstr, jnp.concatenate([esed2, zpad]), whp2, zsrc)

    wop = jnp.concatenate([W_out, jnp.zeros((64, 7), F32)], axis=1)
    bop = jnp.concatenate([b_out, jnp.zeros((7,), F32)]).reshape(1, 8)
    out8 = _stage_post(
        hmid, ud2[:, :N], g2.reshape(1, 64), be2.reshape(1, 64), wop, bop)
    return out8[:, 0]


# speedup vs baseline: 15.4316x; 15.4316x over previous
import functools
import jax
import jax.numpy as jnp
from jax import lax
import jax.scipy.linalg as jsl
from jax.experimental import pallas as pl
from jax.experimental.pallas import tpu as pltpu
from jax.experimental.pallas import tpu_sc as plsc

N = 50000
E = 800000
VOCAB = 1000
N_ACC = 50048
E_PAD = 819200
CH = 1024
NCHUNK = 50
NH = 10000
NQS = 5
H_ACC = 10016
RBH = H_ACC // 16
BN = 1000
F32 = jnp.float32

_sc_mesh = plsc.VectorSubcoreMesh(
    core_axis_name="c", subcore_axis_name="s", num_cores=2, num_subcores=16)


@functools.partial(
    pl.kernel,
    out_type=jax.ShapeDtypeStruct((204800, 16), F32),
    mesh=_sc_mesh,
    compiler_params=pltpu.CompilerParams(use_tc_tiling_on_sc=False, needs_layout_passes=False),
    scratch_types=[
        pltpu.VMEM((128,), jnp.int32),
        pltpu.VMEM((128, 16), F32),
        pltpu.SemaphoreType.DMA,
    ],
)
def _embed(idx2d, tab, out, idxv, rows, sem):
    c = lax.axis_index("c")
    s = lax.axis_index("s")
    wid = s * 2 + c

    def it(r, carry):
        row = wid * 50 + r
        pltpu.sync_copy(idx2d.at[row], idxv)
        pltpu.async_copy(tab.at[idxv], rows, sem).wait()
        pltpu.sync_copy(rows, out.at[pl.ds(row * 128, 128)])
        return carry

    lax.fori_loop(0, 50, it, 0)


@functools.partial(
    pl.kernel,
    out_type=jax.ShapeDtypeStruct((2 * NQS * H_ACC, 34), F32),
    mesh=_sc_mesh,
    compiler_params=pltpu.CompilerParams(use_tc_tiling_on_sc=False, needs_layout_passes=False),
    scratch_types=[
        pltpu.VMEM((8, 128), jnp.int32),
        pltpu.VMEM((8, 128), jnp.int32),
        pltpu.VMEM((8, 128), jnp.int32),
        pltpu.VMEM((8, 128), jnp.int32),
        pltpu.VMEM((CH, 8), F32),
        pltpu.VMEM((CH, 8), F32),
        pltpu.VMEM((CH, 32), F32),
        pltpu.VMEM((CH, 34), F32),
        pltpu.VMEM_SHARED((H_ACC, 34), F32),
        pltpu.SemaphoreType.DMA,
        pltpu.SemaphoreType.DMA,
        pltpu.SemaphoreType.DMA,
    ],
)
def _edge_pass(srcr, dstr, esed, wh2, zsrc, ud,
               idx_s, idx_d, idx_sw, idx_dw, esrows, edrows, whrows, msgal,
               acc, sem_a, sem_b, sem_c):
    c = lax.axis_index("c")
    s = lax.axis_index("s")

    iota = lax.iota(jnp.int32, 16)
    hpar = lax.bitwise_and(iota, 1)
    erow_off = lax.shift_right_logical(iota, 1)
    es_col = 2 * c + hpar
    ed_col = es_col + 4
    al_col = 32 + hpar

    def half_iter(half, hc_):
        lo = half * NH
        pltpu.sync_copy(zsrc, acc.at[pl.ds(s * RBH, RBH)])
        plsc.subcore_barrier()

        def chunk(ch, carry):
            pltpu.sync_copy(srcr.at[pl.ds(s * 400 + ch * 8, 8)], idx_s)
            pltpu.sync_copy(dstr.at[pl.ds(s * 400 + ch * 8, 8)], idx_d)
            for j in range(8):
                for k in range(8):
                    sl = pl.ds(k * 16, 16)
                    dv = idx_d[j, sl] - lo
                    ok = (dv >= 0) & (dv < NH)
                    idx_dw[j, sl] = jnp.where(ok, dv, NH)
                    idx_sw[j, sl] = idx_s[j, sl] + c * N
            descs = []
            for j in range(8):
                descs.append(pltpu.async_copy(
                    esed.at[idx_s.at[j]], esrows.at[pl.ds(j * 128, 128)],
                    sem_a))
                descs.append(pltpu.async_copy(
                    esed.at[idx_d.at[j]], edrows.at[pl.ds(j * 128, 128)],
                    sem_b))
                descs.append(pltpu.async_copy(
                    wh2.at[idx_sw.at[j]], whrows.at[pl.ds(j * 128, 128)],
                    sem_c))
            for d in descs:
                d.wait()

            def group(g, gc):
                rows = g * 8 + erow_off
                es = plsc.load_gather(esrows, [rows, es_col])
                ed = plsc.load_gather(edrows, [rows, ed_col])
                e = es + ed
                e = jnp.maximum(e, 0.2 * e)
                al = jnp.exp(e)
                plsc.store_scatter(msgal, [rows, al_col], al)
                base = g * 8
                for j in range(16):
                    el = base + (j >> 1)
                    col = (j & 1) * 16
                    a = al[j]
                    msgal[el, pl.ds(col, 16)] = whrows[el, pl.ds(col, 16)] * a
                return gc

            lax.fori_loop(0, CH // 8, group, 0)

            for j in range(8):
                pltpu.sync_copy(msgal.at[pl.ds(j * 128, 128)],
                                acc.at[idx_dw.at[j]], add=True)
            return carry

        lax.fori_loop(0, NCHUNK, chunk, 0)

        plsc.subcore_barrier()
        pltpu.sync_copy(
            acc.at[pl.ds(s * RBH, RBH)],
            ud.at[pl.ds((NQS * c + half) * H_ACC + s * RBH, RBH)])
        return hc_

    lax.fori_loop(0, NQS, half_iter, 0)


def _pre_body(xn, cat, wnum, bnum, wina, winb, bin_, w1, ap1,
              h_out, whp, esed_out):
    ne = jnp.maximum(xn[...] @ wnum[...] + bnum[...], 0.)
    h = jnp.maximum(ne @ wina[...] + cat[...] @ winb[...] + bin_[...], 0.)
    h_out[...] = h
    wh = h @ w1[...]
    whp[0] = wh[:, :32]
    whp[1] = wh[:, 32:]
    esed_out[...] = wh @ ap1[...]


def _gat_norm(hv, ud_, g, b):
    parts = []
    for cc in range(2):
        for hh in range(2):
            u = ud_[cc, :, hh * 16:(hh + 1) * 16]
            d = jnp.maximum(ud_[cc, :, 32 + hh:33 + hh], 1e-8)
            parts.append(u / d)
    gat = jnp.concatenate(parts, axis=1)
    y = hv + gat
    m = jnp.mean(y, axis=1, keepdims=True)
    v = jnp.mean((y - m) ** 2, axis=1, keepdims=True)
    yl = (y - m) / jnp.sqrt(v + 1e-5) * g + b
    return jnp.maximum(yl, 0.)


def _mid_body(h, ud, g1, b1, w2, ap2, hmid_out, whp2, esed2):
    hm = _gat_norm(h[...], ud[...], g1[...], b1[...])
    hmid_out[...] = hm
    wh = hm @ w2[...]
    whp2[0] = wh[:, :32]
    whp2[1] = wh[:, 32:]
    esed2[...] = wh @ ap2[...]


def _post_body(h, ud, g2, b2, wop, bop, out):
    hf = _gat_norm(h[...], ud[...], g2[...], b2[...])
    out[...] = hf @ wop[...] + bop[...]


def _wspec(shape):
    nd = len(shape)
    return pl.BlockSpec(shape, lambda i: (0,) * nd)


def _stage_pre(x_num, cat, wnum, bnum, wina, winb, bin_, w1, ap1):
    return pl.pallas_call(
        _pre_body,
        grid=(N // BN,),
        in_specs=[
            pl.BlockSpec((BN, 16), lambda i: (i, 0)),
            pl.BlockSpec((BN, 64), lambda i: (i, 0)),
            _wspec((16, 16)), _wspec((1, 16)), _wspec((16, 64)),
            _wspec((64, 64)), _wspec((1, 64)), _wspec((64, 64)),
            _wspec((64, 8)),
        ],
        out_specs=[
            pl.BlockSpec((BN, 64), lambda i: (i, 0)),
            pl.BlockSpec((2, BN, 32), lambda i: (0, i, 0)),
            pl.BlockSpec((BN, 8), lambda i: (i, 0)),
        ],
        out_shape=[
            jax.ShapeDtypeStruct((N, 64), F32),
            jax.ShapeDtypeStruct((2, N, 32), F32),
            jax.ShapeDtypeStruct((N, 8), F32),
        ],
    )(x_num, cat, wnum, bnum, wina, winb, bin_, w1, ap1)


def _stage_mid(h, ud, g1, b1, w2, ap2):
    return pl.pallas_call(
        _mid_body,
        grid=(N // BN,),
        in_specs=[
            pl.BlockSpec((BN, 64), lambda i: (i, 0)),
            pl.BlockSpec((2, BN, 34), lambda i: (0, i, 0)),
            _wspec((1, 64)), _wspec((1, 64)), _wspec((64, 64)),
            _wspec((64, 8)),
        ],
        out_specs=[
            pl.BlockSpec((BN, 64), lambda i: (i, 0)),
            pl.BlockSpec((2, BN, 32), lambda i: (0, i, 0)),
            pl.BlockSpec((BN, 8), lambda i: (i, 0)),
        ],
        out_shape=[
            jax.ShapeDtypeStruct((N, 64), F32),
            jax.ShapeDtypeStruct((2, N, 32), F32),
            jax.ShapeDtypeStruct((N, 8), F32),
        ],
    )(h, ud, g1, b1, w2, ap2)


def _stage_post(h, ud, g2, b2, wop, bop):
    return pl.pallas_call(
        _post_body,
        grid=(N // BN,),
        in_specs=[
            pl.BlockSpec((BN, 64), lambda i: (i, 0)),
            pl.BlockSpec((2, BN, 34), lambda i: (0, i, 0)),
            _wspec((1, 64)), _wspec((1, 64)), _wspec((64, 8)),
            _wspec((1, 8)),
        ],
        out_specs=pl.BlockSpec((BN, 8), lambda i: (i, 0)),
        out_shape=jax.ShapeDtypeStruct((N, 8), F32),
    )(h, ud, g2, b2, wop, bop)


def _apack(a_src, a_dst):
    bs = jsl.block_diag(*[a_src[h][:, None] for h in range(4)])
    bd = jsl.block_diag(*[a_dst[h][:, None] for h in range(4)])
    return jnp.concatenate([bs, bd], axis=1)


@jax.jit
def kernel(x_num, x_cat, edge_index, W_num, b_num, E0, E1, E2, E3, W_in, b_in,
           W1, a_src1, a_dst1, W2, a_src2, a_dst2, g1, be1, g2, be2,
           W_out, b_out):
    x_num = x_num.astype(F32)
    xc = jnp.clip(x_cat, 0, VOCAB).astype(jnp.int32)
    idx_flat = (xc + jnp.arange(4, dtype=jnp.int32) * (VOCAB + 1)).reshape(-1)
    idx_pad = jnp.concatenate(
        [idx_flat, jnp.zeros((204800 - 4 * N,), jnp.int32)]).reshape(1600, 128)
    Ecat = jnp.concatenate([E0, E1, E2, E3], axis=0)
    emb = _embed(idx_pad, Ecat)
    cat_embed = emb[:4 * N].reshape(N, 64)

    src = edge_index[0].astype(jnp.int32)
    dst = edge_index[1].astype(jnp.int32)
    pe = E_PAD - E
    srcr = jnp.concatenate([src, jnp.zeros((pe,), jnp.int32)]).reshape(6400, 128)
    dstr = jnp.concatenate([dst, jnp.full((pe,), N, jnp.int32)]).reshape(6400, 128)
    zsrc = jnp.zeros((RBH, 34), F32)
    zpad = jnp.zeros((N_ACC - N, 8), F32)

    def _ud_assemble(ud_flat):
        return ud_flat.reshape(2, NQS, H_ACC, 34)[:, :, :NH, :].reshape(2, N, 34)

    ap1 = _apack(a_src1, a_dst1)
    ap2 = _apack(a_src2, a_dst2)

    h, whp1, esed1 = _stage_pre(
        x_num, cat_embed, W_num, b_num.reshape(1, 16), W_in[:16], W_in[16:],
        b_in.reshape(1, 64), W1, ap1)
    ud1 = _edge_pass(srcr, dstr, jnp.concatenate([esed1, zpad]),
                     whp1.reshape(2 * N, 32), zsrc)
    hmid, whp2, esed2 = _stage_mid(
        h, _ud_assemble(ud1), g1.reshape(1, 64), be1.reshape(1, 64), W2, ap2)
    ud2 = _edge_pass(srcr, dstr, jnp.concatenate([esed2, zpad]),
                     whp2.reshape(2 * N, 32), zsrc)

    wop = jnp.concatenate([W_out, jnp.zeros((64, 7), F32)], axis=1)
    bop = jnp.concatenate([b_out, jnp.zeros((7,), F32)]).reshape(1, 8)
    out8 = _stage_post(
        hmid, _ud_assemble(ud2), g2.reshape(1, 64), be2.reshape(1, 64), wop,
        bop)
    return out8[:, 0]
